# baseline (speedup 1.0000x reference)
import functools

import jax
import jax.numpy as jnp
from jax import lax
from jax.experimental import pallas as pl
from jax.experimental.pallas import tpu as pltpu
from jax.experimental.pallas import tpu_sc as plsc

_NC = 2
_NS = 16
_NW = _NC * _NS
_LANES = 16
_DMA_ROWS = 128


def _pooled_context_sc(context, emb_table):
    B, CTX = context.shape
    V, E = emb_table.shape
    rows_per_worker = (B * CTX) // _NW
    elems_per_worker = B // _NW
    n_chunks = 2
    rows_per_chunk = rows_per_worker // n_chunks
    elems_per_chunk = elems_per_worker // n_chunks
    dmas_per_chunk = rows_per_chunk // _DMA_ROWS
    assert rows_per_chunk % _DMA_ROWS == 0
    assert elems_per_chunk * CTX == rows_per_chunk
    assert E % _LANES == 0

    ctx_r = context.reshape(_NW, n_chunks, dmas_per_chunk, _DMA_ROWS)

    mesh = plsc.VectorSubcoreMesh(core_axis_name="c", subcore_axis_name="s")

    @functools.partial(
        pl.kernel,
        out_type=jax.ShapeDtypeStruct((B, E), jnp.float32),
        mesh=mesh,
        scratch_types=[
            pltpu.VMEM((dmas_per_chunk, _DMA_ROWS), jnp.int32),
            pltpu.VMEM((rows_per_chunk, E), jnp.float32),
            pltpu.VMEM((elems_per_chunk, E), jnp.float32),
            pltpu.SemaphoreType.DMA,
        ],
        compiler_params=pltpu.CompilerParams(use_tc_tiling_on_sc=False),
    )
    def gather_mean(ctx_hbm, table_hbm, cv_hbm, idx_v, rows_v, out_v, sem):
        w = lax.axis_index("s") * _NC + lax.axis_index("c")
        inv = jnp.float32(1.0 / CTX)
        for ch in range(n_chunks):
            pltpu.sync_copy(ctx_hbm.at[w, ch], idx_v)
            copies = []
            for j in range(dmas_per_chunk):
                copies.append(
                    pltpu.async_copy(
                        table_hbm.at[idx_v.at[j]],
                        rows_v.at[pl.ds(j * _DMA_ROWS, _DMA_ROWS)],
                        sem,
                    )
                )
            for cp in copies:
                cp.wait()

            def pool_one(e, carry):
                base = e * CTX
                for d in range(E // _LANES):
                    sl = pl.ds(d * _LANES, _LANES)
                    acc = rows_v[base, sl]
                    for t in range(1, CTX):
                        acc = acc + rows_v[base + t, sl]
                    out_v[e, sl] = acc * inv
                return carry

            lax.fori_loop(0, elems_per_chunk, pool_one, 0)
            pltpu.sync_copy(
                out_v,
                cv_hbm.at[
                    pl.ds(w * elems_per_worker + ch * elems_per_chunk,
                          elems_per_chunk)
                ],
            )

    return gather_mean(ctx_r, emb_table)


def _pooled_context_sc_v2(contextT, tableT):
    CTX, B = contextT.shape
    E, V = tableT.shape
    dims_per_worker = E // _NW
    n_chunks = 8
    BC = B // n_chunks
    n_groups = BC // _LANES
    assert dims_per_worker * _NW == E and BC * n_chunks == B

    mesh = plsc.VectorSubcoreMesh(core_axis_name="c", subcore_axis_name="s")

    @functools.partial(
        pl.kernel,
        out_type=jax.ShapeDtypeStruct((E, B), jnp.float32),
        mesh=mesh,
        scratch_types=[
            pltpu.VMEM((V,), jnp.float32),
            pltpu.VMEM((2, CTX, BC), jnp.int32),
            pltpu.VMEM((2, BC), jnp.float32),
            pltpu.SemaphoreType.DMA,
            pltpu.SemaphoreType.DMA,
        ],
        compiler_params=pltpu.CompilerParams(
            use_tc_tiling_on_sc=True, needs_layout_passes=False,
        ),
    )
    def gather_mean(ctx_hbm, table_hbm, cvt_hbm, row_v, idx_v, out_v,
                    sem_ctx, sem_row):
        w = lax.axis_index("s") * _NC + lax.axis_index("c")
        inv = jnp.float32(1.0 / CTX)
        for k in range(dims_per_worker):
            d = w * dims_per_worker + k
            row_cp = pltpu.async_copy(table_hbm.at[d], row_v, sem_row)
            pltpu.async_copy(ctx_hbm.at[:, pl.ds(0, BC)], idx_v.at[0], sem_ctx)
            row_cp.wait()

            def pair(i, carry):
                for bslot in range(2):
                    ch = 2 * i + bslot
                    nxt = (ch + 1) % n_chunks
                    pltpu.async_copy(
                        ctx_hbm.at[:, pl.ds(nxt * BC, BC)],
                        idx_v.at[(bslot + 1) % 2],
                        sem_ctx,
                    )
                    pltpu.make_async_copy(
                        ctx_hbm.at[:, pl.ds(0, BC)], idx_v.at[0], sem_ctx
                    ).wait()
                    for g in range(n_groups):
                        sl = pl.ds(g * _LANES, _LANES)
                        acc = plsc.load_gather(row_v, [idx_v[bslot, 0, sl]])
                        for c in range(1, CTX):
                            acc = acc + plsc.load_gather(
                                row_v, [idx_v[bslot, c, sl]])
                        out_v[bslot, sl] = acc * inv
                    pltpu.sync_copy(
                        out_v.at[bslot], cvt_hbm.at[d, pl.ds(ch * BC, BC)])
                return carry

            lax.fori_loop(0, n_chunks // 2, pair, 0)
            pltpu.make_async_copy(
                ctx_hbm.at[:, pl.ds(0, BC)], idx_v.at[0], sem_ctx
            ).wait()

    return gather_mean(contextT, tableT)


def _project_tc(cvT, Wt, b, VB=1024):
    E, B = cvT.shape
    V = Wt.shape[1]
    nv = pl.cdiv(V, VB)

    def body(cvt_ref, wt_ref, b_ref, out_ref):
        acc = lax.dot_general(
            wt_ref[...].astype(jnp.bfloat16), cvt_ref[...].astype(jnp.bfloat16),
            dimension_numbers=(((0,), (0,)), ((), ())),
            preferred_element_type=jnp.float32,
        )
        out_ref[...] = acc + b_ref[...][:, None]

    out_t = pl.pallas_call(
        body,
        grid=(nv,),
        in_specs=[
            pl.BlockSpec((E, B), lambda j: (0, 0)),
            pl.BlockSpec((E, VB), lambda j: (0, j)),
            pl.BlockSpec((VB,), lambda j: (j,)),
        ],
        out_specs=pl.BlockSpec((VB, B), lambda j: (j, 0)),
        out_shape=jax.ShapeDtypeStruct((V, B), jnp.float32),
        compiler_params=pltpu.CompilerParams(
            dimension_semantics=("arbitrary",),
        ),
    )(cvT, Wt, b)
    return out_t.T


def kernel(context, emb_table, W, b):
    cvT = _pooled_context_sc_v2(context.T, emb_table.T)
    return _project_tc(cvT, W.T, b)


# speedup vs baseline: 1.0073x; 1.0073x over previous
import functools

import jax
import jax.numpy as jnp
from jax import lax
from jax.experimental import pallas as pl
from jax.experimental.pallas import tpu as pltpu
from jax.experimental.pallas import tpu_sc as plsc

_NC = 2
_NS = 16
_NW = _NC * _NS
_LANES = 16
_DMA_ROWS = 128


def _pooled_context_sc(context, emb_table):
    B, CTX = context.shape
    V, E = emb_table.shape
    rows_per_worker = (B * CTX) // _NW
    elems_per_worker = B // _NW
    n_chunks = 2
    rows_per_chunk = rows_per_worker // n_chunks
    elems_per_chunk = elems_per_worker // n_chunks
    dmas_per_chunk = rows_per_chunk // _DMA_ROWS
    assert rows_per_chunk % _DMA_ROWS == 0
    assert elems_per_chunk * CTX == rows_per_chunk
    assert E % _LANES == 0

    ctx_r = context.reshape(_NW, n_chunks, dmas_per_chunk, _DMA_ROWS)

    mesh = plsc.VectorSubcoreMesh(core_axis_name="c", subcore_axis_name="s")

    @functools.partial(
        pl.kernel,
        out_type=jax.ShapeDtypeStruct((B, E), jnp.float32),
        mesh=mesh,
        scratch_types=[
            pltpu.VMEM((dmas_per_chunk, _DMA_ROWS), jnp.int32),
            pltpu.VMEM((rows_per_chunk, E), jnp.float32),
            pltpu.VMEM((elems_per_chunk, E), jnp.float32),
            pltpu.SemaphoreType.DMA,
        ],
        compiler_params=pltpu.CompilerParams(use_tc_tiling_on_sc=False),
    )
    def gather_mean(ctx_hbm, table_hbm, cv_hbm, idx_v, rows_v, out_v, sem):
        w = lax.axis_index("s") * _NC + lax.axis_index("c")
        inv = jnp.float32(1.0 / CTX)
        for ch in range(n_chunks):
            pltpu.sync_copy(ctx_hbm.at[w, ch], idx_v)
            copies = []
            for j in range(dmas_per_chunk):
                copies.append(
                    pltpu.async_copy(
                        table_hbm.at[idx_v.at[j]],
                        rows_v.at[pl.ds(j * _DMA_ROWS, _DMA_ROWS)],
                        sem,
                    )
                )
            for cp in copies:
                cp.wait()

            def pool_one(e, carry):
                base = e * CTX
                for d in range(E // _LANES):
                    sl = pl.ds(d * _LANES, _LANES)
                    acc = rows_v[base, sl]
                    for t in range(1, CTX):
                        acc = acc + rows_v[base + t, sl]
                    out_v[e, sl] = acc * inv
                return carry

            lax.fori_loop(0, elems_per_chunk, pool_one, 0)
            pltpu.sync_copy(
                out_v,
                cv_hbm.at[
                    pl.ds(w * elems_per_worker + ch * elems_per_chunk,
                          elems_per_chunk)
                ],
            )

    return gather_mean(ctx_r, emb_table)


def _pooled_context_sc_v2(contextT, tableT):
    CTX, B = contextT.shape
    E, V = tableT.shape
    dims_per_worker = E // _NW
    n_chunks = 8
    BC = B // n_chunks
    n_groups = BC // _LANES
    assert dims_per_worker * _NW == E and BC * n_chunks == B

    mesh = plsc.VectorSubcoreMesh(core_axis_name="c", subcore_axis_name="s")

    @functools.partial(
        pl.kernel,
        out_type=jax.ShapeDtypeStruct((E, B), jnp.float32),
        mesh=mesh,
        scratch_types=[
            pltpu.VMEM((V,), jnp.float32),
            pltpu.VMEM((2, CTX, BC), jnp.int32),
            pltpu.VMEM((2, BC), jnp.float32),
            pltpu.SemaphoreType.DMA,
            pltpu.SemaphoreType.DMA,
        ],
        compiler_params=pltpu.CompilerParams(
            use_tc_tiling_on_sc=True, needs_layout_passes=False,
        ),
    )
    def gather_mean(ctx_hbm, table_hbm, cvt_hbm, row_v, idx_v, out_v,
                    sem_ctx, sem_row):
        w = lax.axis_index("s") * _NC + lax.axis_index("c")
        inv = jnp.float32(1.0 / CTX)
        for k in range(dims_per_worker):
            d = w * dims_per_worker + k
            row_cp = pltpu.async_copy(table_hbm.at[d], row_v, sem_row)
            pltpu.async_copy(ctx_hbm.at[:, pl.ds(0, BC)], idx_v.at[0], sem_ctx)
            row_cp.wait()

            def pair(i, carry):
                for bslot in range(2):
                    ch = 2 * i + bslot
                    nxt = (ch + 1) % n_chunks
                    pltpu.async_copy(
                        ctx_hbm.at[:, pl.ds(nxt * BC, BC)],
                        idx_v.at[(bslot + 1) % 2],
                        sem_ctx,
                    )
                    pltpu.make_async_copy(
                        ctx_hbm.at[:, pl.ds(0, BC)], idx_v.at[0], sem_ctx
                    ).wait()
                    for g in range(n_groups):
                        sl = pl.ds(g * _LANES, _LANES)
                        accs = [
                            plsc.load_gather(row_v, [idx_v[bslot, c0, sl]])
                            for c0 in range(4)
                        ]
                        for c in range(4, CTX):
                            accs[c % 4] = accs[c % 4] + plsc.load_gather(
                                row_v, [idx_v[bslot, c, sl]])
                        out_v[bslot, sl] = (
                            (accs[0] + accs[1]) + (accs[2] + accs[3])
                        ) * inv
                    pltpu.sync_copy(
                        out_v.at[bslot], cvt_hbm.at[d, pl.ds(ch * BC, BC)])
                return carry

            lax.fori_loop(0, n_chunks // 2, pair, 0)
            pltpu.make_async_copy(
                ctx_hbm.at[:, pl.ds(0, BC)], idx_v.at[0], sem_ctx
            ).wait()

    return gather_mean(contextT, tableT)


def _project_tc(cvT, Wt, b, VB=1024):
    E, B = cvT.shape
    V = Wt.shape[1]
    nv = pl.cdiv(V, VB)

    def body(cvt_ref, wt_ref, b_ref, out_ref):
        acc = lax.dot_general(
            wt_ref[...].astype(jnp.bfloat16), cvt_ref[...].astype(jnp.bfloat16),
            dimension_numbers=(((0,), (0,)), ((), ())),
            preferred_element_type=jnp.float32,
        )
        out_ref[...] = acc + b_ref[...][:, None]

    out_t = pl.pallas_call(
        body,
        grid=(nv,),
        in_specs=[
            pl.BlockSpec((E, B), lambda j: (0, 0)),
            pl.BlockSpec((E, VB), lambda j: (0, j)),
            pl.BlockSpec((VB,), lambda j: (j,)),
        ],
        out_specs=pl.BlockSpec((VB, B), lambda j: (j, 0)),
        out_shape=jax.ShapeDtypeStruct((V, B), jnp.float32),
        compiler_params=pltpu.CompilerParams(
            dimension_semantics=("arbitrary",),
        ),
    )(cvT, Wt, b)
    return out_t.T


def kernel(context, emb_table, W, b):
    cvT = _pooled_context_sc_v2(context.T, emb_table.T)
    return _project_tc(cvT, W.T, b)
